# baseline (speedup 1.0000x reference)
import dataclasses

import jax
import jax.numpy as jnp
from jax import lax
from jax.experimental import pallas as pl
from jax.experimental.pallas import tpu as pltpu
from jax.experimental.pallas import tpu_sc as plsc

_NC, _NS, _L = 2, 16, 16
_NW = _NC * _NS
_BATCH = 16384
_D = 64
_BW = _BATCH // _NW
_CH = 128
_NCH = _BW // _CH


def _vsqrt(x):
    i = plsc.bitcast(x, jnp.int32)
    y = plsc.bitcast(jnp.int32(0x5F3759DF) - (i >> 1), jnp.float32)
    for _ in range(3):
        y = y * (1.5 - 0.5 * x * y * y)
    return x * y


def _body(heads_hbm, rels_hbm, tails_hbm, ent_hbm, rel_hbm, out_hbm,
          hidx_v, ridx_v, tidx_v, hpair_v, rpair_v, tpair_v,
          hhalf_v, rhalf_v, thalf_v, h_v, r_v, t_v, sq_v, s_v, sem):
    wid = lax.axis_index("s") * _NC + lax.axis_index("c")
    base = wid * _BW
    pltpu.sync_copy(heads_hbm.at[pl.ds(base, _BW)], hidx_v)
    pltpu.sync_copy(rels_hbm.at[pl.ds(base, _BW)], ridx_v)
    pltpu.sync_copy(tails_hbm.at[pl.ds(base, _BW)], tidx_v)

    @pl.loop(0, _BW, step=_L)
    def _split(i):
        sl = pl.ds(i, _L)
        for idx_v, pair_v, half_v in ((hidx_v, hpair_v, hhalf_v),
                                      (ridx_v, rpair_v, rhalf_v),
                                      (tidx_v, tpair_v, thalf_v)):
            v = idx_v[sl]
            pair_v[sl] = v >> 1
            half_v[sl] = (v & 1) * _D

    lanes = lax.iota(jnp.int32, _L)

    @pl.loop(0, _NCH)
    def _chunk(c):
        off = c * _CH
        ch = pltpu.async_copy(ent_hbm.at[hpair_v.at[pl.ds(off, _CH)]], h_v, sem)
        cr = pltpu.async_copy(rel_hbm.at[rpair_v.at[pl.ds(off, _CH)]], r_v, sem)
        ct = pltpu.async_copy(ent_hbm.at[tpair_v.at[pl.ds(off, _CH)]], t_v, sem)
        ch.wait()
        cr.wait()
        ct.wait()

        @pl.loop(0, _CH)
        def _row(i):
            hh = plsc.load_gather(hhalf_v, [jnp.full((_L,), off + i, jnp.int32)])
            rh = plsc.load_gather(rhalf_v, [jnp.full((_L,), off + i, jnp.int32)])
            th = plsc.load_gather(thalf_v, [jnp.full((_L,), off + i, jnp.int32)])
            irow = jnp.full((_L,), i, jnp.int32)
            acc = jnp.zeros((_L,), jnp.float32)
            for j in range(_D // _L):
                cvec = j * _L + lanes
                hv = plsc.load_gather(h_v, [irow, hh + cvec])
                rv = plsc.load_gather(r_v, [irow, rh + cvec])
                tv = plsc.load_gather(t_v, [irow, th + cvec])
                d = hv + rv - tv
                acc = acc + d * d
            sq_v[i, :] = acc

        @pl.loop(0, _CH, step=_L)
        def _grp(i0):
            rows = i0 + lanes
            tot = jnp.zeros((_L,), jnp.float32)
            for col in range(_L):
                cols = jnp.full((_L,), col, jnp.int32)
                tot = tot + plsc.load_gather(sq_v, [rows, cols])
            s_v[pl.ds(i0, _L)] = _vsqrt(tot)

        pltpu.sync_copy(s_v, out_hbm.at[pl.ds(base + off, _CH)])


@jax.jit
def kernel(heads, relations, tails, entity_emb, relation_emb):
    ent2 = entity_emb.reshape(entity_emb.shape[0] // 2, 2 * _D)
    rel2 = relation_emb.reshape(relation_emb.shape[0] // 2, 2 * _D)
    mesh = plsc.VectorSubcoreMesh(core_axis_name="c", subcore_axis_name="s")
    cp = pltpu.CompilerParams()
    if "needs_layout_passes" in pltpu.CompilerParams.__dataclass_fields__:
        cp = dataclasses.replace(cp, needs_layout_passes=False)
    if "use_tc_tiling_on_sc" in pltpu.CompilerParams.__dataclass_fields__:
        cp = dataclasses.replace(cp, use_tc_tiling_on_sc=False)
    run = pl.kernel(
        _body,
        out_type=jax.ShapeDtypeStruct((_BATCH,), jnp.float32),
        mesh=mesh,
        scratch_types=[
            pltpu.VMEM((_BW,), jnp.int32),
            pltpu.VMEM((_BW,), jnp.int32),
            pltpu.VMEM((_BW,), jnp.int32),
            pltpu.VMEM((_BW,), jnp.int32),
            pltpu.VMEM((_BW,), jnp.int32),
            pltpu.VMEM((_BW,), jnp.int32),
            pltpu.VMEM((_BW,), jnp.int32),
            pltpu.VMEM((_BW,), jnp.int32),
            pltpu.VMEM((_BW,), jnp.int32),
            pltpu.VMEM((_CH, 2 * _D), jnp.float32),
            pltpu.VMEM((_CH, 2 * _D), jnp.float32),
            pltpu.VMEM((_CH, 2 * _D), jnp.float32),
            pltpu.VMEM((_CH, _L), jnp.float32),
            pltpu.VMEM((_CH,), jnp.float32),
            pltpu.SemaphoreType.DMA,
        ],
        compiler_params=cp,
    )
    return run(heads, relations, tails, ent2, rel2)


# speedup vs baseline: 1.7193x; 1.7193x over previous
import dataclasses

import jax
import jax.numpy as jnp
from jax import lax
from jax.experimental import pallas as pl
from jax.experimental.pallas import tpu as pltpu
from jax.experimental.pallas import tpu_sc as plsc

_NC, _NS, _L = 2, 16, 16
_NW = _NC * _NS
_BATCH = 16384
_D = 64
_BW = _BATCH // _NW
_CH = 128
_NCH = _BW // _CH


def _vsqrt(x):
    i = plsc.bitcast(x, jnp.int32)
    y = plsc.bitcast(jnp.int32(0x5F3759DF) - (i >> 1), jnp.float32)
    for _ in range(3):
        y = y * (1.5 - 0.5 * x * y * y)
    return x * y


def _body(heads_hbm, rels_hbm, tails_hbm, ent_hbm, rel_hbm, out_hbm,
          idx_sp, hidx_s, ridx_s, tidx_s, h_v, r_v, t_v, sq_v, s_v, sem):
    wid = lax.axis_index("s") * _NC + lax.axis_index("c")
    sid = lax.axis_index("s")
    base = wid * _BW
    sb = sid * 3 * _BW
    pltpu.sync_copy(heads_hbm.at[pl.ds(base, _BW)], idx_sp.at[pl.ds(sb, _BW)])
    pltpu.sync_copy(rels_hbm.at[pl.ds(base, _BW)], idx_sp.at[pl.ds(sb + _BW, _BW)])
    pltpu.sync_copy(tails_hbm.at[pl.ds(base, _BW)], idx_sp.at[pl.ds(sb + 2 * _BW, _BW)])
    pltpu.sync_copy(idx_sp.at[pl.ds(sb, _BW)], hidx_s)
    pltpu.sync_copy(idx_sp.at[pl.ds(sb + _BW, _BW)], ridx_s)
    pltpu.sync_copy(idx_sp.at[pl.ds(sb + 2 * _BW, _BW)], tidx_s)

    lanes = lax.iota(jnp.int32, _L)

    @pl.loop(0, _NCH)
    def _chunk(c):
        off = c * _CH

        @pl.loop(0, _CH)
        def _fire(i):
            pltpu.async_copy(ent_hbm.at[pl.ds(hidx_s[off + i], 1)], h_v.at[pl.ds(i, 1)], sem)
            pltpu.async_copy(rel_hbm.at[pl.ds(ridx_s[off + i], 1)], r_v.at[pl.ds(i, 1)], sem)
            pltpu.async_copy(ent_hbm.at[pl.ds(tidx_s[off + i], 1)], t_v.at[pl.ds(i, 1)], sem)

        @pl.loop(0, _CH)
        def _drain(i):
            pltpu.make_async_copy(ent_hbm.at[pl.ds(hidx_s[off + i], 1)], h_v.at[pl.ds(i, 1)], sem).wait()
            pltpu.make_async_copy(rel_hbm.at[pl.ds(ridx_s[off + i], 1)], r_v.at[pl.ds(i, 1)], sem).wait()
            pltpu.make_async_copy(ent_hbm.at[pl.ds(tidx_s[off + i], 1)], t_v.at[pl.ds(i, 1)], sem).wait()

        @pl.loop(0, _CH)
        def _row(i):
            acc = jnp.zeros((_L,), jnp.float32)
            for j in range(_D // _L):
                sl = pl.ds(j * _L, _L)
                d = h_v[i, sl] + r_v[i, sl] - t_v[i, sl]
                acc = acc + d * d
            sq_v[i, :] = acc

        @pl.loop(0, _CH, step=_L)
        def _grp(i0):
            rows = i0 + lanes
            tot = jnp.zeros((_L,), jnp.float32)
            for col in range(_L):
                cols = jnp.full((_L,), col, jnp.int32)
                tot = tot + plsc.load_gather(sq_v, [rows, cols])
            s_v[pl.ds(i0, _L)] = _vsqrt(tot)

        pltpu.sync_copy(s_v, out_hbm.at[pl.ds(base + off, _CH)])


@jax.jit
def kernel(heads, relations, tails, entity_emb, relation_emb):
    mesh = plsc.VectorSubcoreMesh(core_axis_name="c", subcore_axis_name="s")
    cp = pltpu.CompilerParams()
    if "needs_layout_passes" in pltpu.CompilerParams.__dataclass_fields__:
        cp = dataclasses.replace(cp, needs_layout_passes=False)
    run = pl.kernel(
        _body,
        out_type=jax.ShapeDtypeStruct((_BATCH,), jnp.float32),
        mesh=mesh,
        scratch_types=[
            pltpu.VMEM_SHARED((_NS * 3 * _BW,), jnp.int32),
            pltpu.SMEM((_BW,), jnp.int32),
            pltpu.SMEM((_BW,), jnp.int32),
            pltpu.SMEM((_BW,), jnp.int32),
            pltpu.VMEM((_CH, _D), jnp.float32),
            pltpu.VMEM((_CH, _D), jnp.float32),
            pltpu.VMEM((_CH, _D), jnp.float32),
            pltpu.VMEM((_CH, _L), jnp.float32),
            pltpu.VMEM((_CH,), jnp.float32),
            pltpu.SemaphoreType.DMA,
        ],
        compiler_params=cp,
    )
    return run(heads, relations, tails, entity_emb, relation_emb)
